# baseline (speedup 1.0000x reference)
import functools

import jax
import jax.numpy as jnp
from jax import lax
from jax.experimental import pallas as pl
from jax.experimental.pallas import tpu as pltpu
from jax.experimental.pallas import tpu_sc as plsc

NC = 2
NS = 16
NW = NC * NS
CHUNK = 128
GP = 8
POS_ROWS = 400
POS_Q = 8

GRP_ROWS = GP * CHUNK


def _do_table(wid, idx_hbm, tab, out_hbm, idx_v, rows, sem_g, sem_s, nch, per_w):
    ngrp = nch // GP
    pltpu.sync_copy(idx_hbm.at[wid], idx_v)

    def fire(g, h):
        for j in range(GP):
            pltpu.async_copy(
                tab.at[idx_v.at[g * GP + j]],
                rows.at[pl.ds(h * GRP_ROWS + j * CHUNK, CHUNK)],
                sem_g,
            )

    def wait_gathers(h):
        for j in range(GP):
            pltpu.make_async_copy(
                tab.at[idx_v.at[j]],
                rows.at[pl.ds(h * GRP_ROWS + j * CHUNK, CHUNK)],
                sem_g,
            ).wait()

    def store(g, h):
        base = wid * per_w + g * GRP_ROWS
        return pltpu.async_copy(
            rows.at[pl.ds(h * GRP_ROWS, GRP_ROWS)],
            out_hbm.at[pl.ds(base, GRP_ROWS)],
            sem_s,
        )

    def wait_store(g, h):
        base = wid * per_w + g * GRP_ROWS
        pltpu.make_async_copy(
            rows.at[pl.ds(h * GRP_ROWS, GRP_ROWS)],
            out_hbm.at[pl.ds(base, GRP_ROWS)],
            sem_s,
        ).wait()

    fire(0, 0)
    fire(1, 1)

    def body(g, carry):
        h = g % 2
        wait_gathers(h)
        store(g, h)
        wait_store(g, h)
        fire_g = g + 2
        for j in range(GP):
            pltpu.async_copy(
                tab.at[idx_v.at[fire_g * GP + j]],
                rows.at[pl.ds(h * GRP_ROWS + j * CHUNK, CHUNK)],
                sem_g,
            )
        return carry

    lax.fori_loop(0, ngrp - 2, body, 0)

    for g in (ngrp - 2, ngrp - 1):
        h = g % 2
        wait_gathers(h)
        store(g, h)
    for g in (ngrp - 2, ngrp - 1):
        wait_store(g, g % 2)


def kernel(qids, uids, vids, clicks, Q_table, U_table, C_table, V_table, P_table):
    B, L = qids.shape
    N = B * L
    per_w = N // NW
    nch = per_w // CHUNK
    E = Q_table.shape[1]
    CE = C_table.shape[1]

    qi = qids.reshape(NW, nch, CHUNK)
    ui = uids.reshape(NW, nch, CHUNK)
    vi = vids.reshape(NW, nch, CHUNK)
    ci = clicks.reshape(NW, nch, CHUNK)

    mesh = plsc.VectorSubcoreMesh(core_axis_name="c", subcore_axis_name="s")

    @functools.partial(
        pl.kernel,
        mesh=mesh,
        compiler_params=pltpu.CompilerParams(use_tc_tiling_on_sc=False),
        out_type=[
            jax.ShapeDtypeStruct((N, E), jnp.float32),
            jax.ShapeDtypeStruct((N, E), jnp.float32),
            jax.ShapeDtypeStruct((N, CE), jnp.float32),
            jax.ShapeDtypeStruct((N, CE), jnp.float32),
            jax.ShapeDtypeStruct((N, CE), jnp.float32),
        ],
        scratch_types=[
            pltpu.VMEM((nch, CHUNK), jnp.int32),
            pltpu.VMEM((2 * GRP_ROWS, E), jnp.float32),
            pltpu.VMEM((2 * GRP_ROWS, CE), jnp.float32),
            pltpu.VMEM((POS_ROWS, CE), jnp.float32),
            pltpu.VMEM((10, CE), jnp.float32),
            pltpu.SemaphoreType.DMA,
            pltpu.SemaphoreType.DMA,
        ],
    )
    def k(qi_h, ui_h, vi_h, ci_h, Qt, Ut, Ct, Vt, Pt,
          oq, ou, oc, ov, opos, idx_v, r32, r16, posb, pv, sem_g, sem_s):
        wid = lax.axis_index("s") * NC + lax.axis_index("c")

        with jax.named_scope("q_gather"):
            _do_table(wid, qi_h, Qt, oq, idx_v, r32, sem_g, sem_s, nch, per_w)
        with jax.named_scope("u_gather"):
            _do_table(wid, ui_h, Ut, ou, idx_v, r32, sem_g, sem_s, nch, per_w)
        with jax.named_scope("c_gather"):
            _do_table(wid, ci_h, Ct, oc, idx_v, r16, sem_g, sem_s, nch, per_w)
        with jax.named_scope("v_gather"):
            _do_table(wid, vi_h, Vt, ov, idx_v, r16, sem_g, sem_s, nch, per_w)

        pltpu.sync_copy(Pt, pv)
        prow = [pv[i, :] for i in range(10)]
        for b in range(POS_ROWS // 10):
            for r in range(10):
                posb[b * 10 + r, :] = prow[r]

        nstores = per_w // POS_ROWS

        def pos_body(t, carry):
            cps = []
            for u in range(POS_Q):
                base = wid * per_w + (t * POS_Q + u) * POS_ROWS
                cps.append(
                    pltpu.async_copy(
                        posb, opos.at[pl.ds(base, POS_ROWS)], sem_s
                    )
                )
            for cp in cps:
                cp.wait()
            return carry

        with jax.named_scope("pos_store"):
            lax.fori_loop(0, nstores // POS_Q, pos_body, 0)

    oq, ou, oc, ov, opos = k(qi, ui, vi, ci, Q_table, U_table, C_table, V_table, P_table)
    return (
        oq.reshape(B, L, E),
        ou.reshape(B, L, E),
        oc.reshape(B, L, CE),
        ov.reshape(B, L, CE),
        opos.reshape(B, L, CE),
    )


# speedup vs baseline: 2.5391x; 2.5391x over previous
import functools

import jax
import jax.numpy as jnp
from jax import lax
from jax.experimental import pallas as pl
from jax.experimental.pallas import tpu as pltpu
from jax.experimental.pallas import tpu_sc as plsc

NC = 2
NS = 16
NW = NC * NS
NSUB = 2
GP = 8
POS_ROWS = 400
POS_Q = 8


def _do_table(wid, idx_hbm, tab, out_hbm, idxbuf, rows, sem_g, sem_s,
              rows_w, L, per_w):
    subs = [(0, 128), (128, L - 128)]
    GRP = (GP // NSUB) * L
    ngrp = per_w // GRP
    rpg = GP // NSUB

    pltpu.sync_copy(idx_hbm.at[pl.ds(wid * rows_w, rows_w)], idxbuf)

    def fire(g, h):
        for j in range(GP):
            r = g * rpg + j // NSUB
            off, sz = subs[j % NSUB]
            dst = (j // NSUB) * L + off
            pltpu.async_copy(
                tab.at[idxbuf.at[r, pl.ds(off, sz)]],
                rows.at[pl.ds(h * GRP + dst, sz)],
                sem_g,
            )

    def wait_gathers(h):
        for j in range(GP):
            off, sz = subs[j % NSUB]
            dst = (j // NSUB) * L + off
            pltpu.make_async_copy(
                tab.at[idxbuf.at[0, pl.ds(off, sz)]],
                rows.at[pl.ds(h * GRP + dst, sz)],
                sem_g,
            ).wait()

    def store(g, h):
        pltpu.async_copy(
            rows.at[pl.ds(h * GRP, GRP)],
            out_hbm.at[pl.ds(wid * per_w + g * GRP, GRP)],
            sem_s,
        )

    def wait_store(g, h):
        pltpu.make_async_copy(
            rows.at[pl.ds(h * GRP, GRP)],
            out_hbm.at[pl.ds(wid * per_w + g * GRP, GRP)],
            sem_s,
        ).wait()

    fire(0, 0)
    fire(1, 1)

    def body(g, carry):
        h = g % 2
        wait_gathers(h)
        store(g, h)
        wait_store(g, h)
        fire(g + 2, h)
        return carry

    lax.fori_loop(0, ngrp - 2, body, 0)

    for g in (ngrp - 2, ngrp - 1):
        wait_gathers(g % 2)
        store(g, g % 2)
    for g in (ngrp - 2, ngrp - 1):
        wait_store(g, g % 2)


def kernel(qids, uids, vids, clicks, Q_table, U_table, C_table, V_table, P_table):
    B, L = qids.shape
    N = B * L
    per_w = N // NW
    rows_w = B // NW
    E = Q_table.shape[1]
    CE = C_table.shape[1]
    GRP = (GP // NSUB) * L

    mesh = plsc.VectorSubcoreMesh(core_axis_name="c", subcore_axis_name="s")

    @functools.partial(
        pl.kernel,
        mesh=mesh,
        compiler_params=pltpu.CompilerParams(use_tc_tiling_on_sc=False),
        out_type=[
            jax.ShapeDtypeStruct((N, E), jnp.float32),
            jax.ShapeDtypeStruct((N, E), jnp.float32),
            jax.ShapeDtypeStruct((N, CE), jnp.float32),
            jax.ShapeDtypeStruct((N, CE), jnp.float32),
            jax.ShapeDtypeStruct((N, CE), jnp.float32),
        ],
        scratch_types=[
            pltpu.VMEM((rows_w, L), jnp.int32),
            pltpu.VMEM((2 * GRP, E), jnp.float32),
            pltpu.VMEM((2 * GRP, CE), jnp.float32),
            pltpu.VMEM((POS_ROWS, CE), jnp.float32),
            pltpu.VMEM((10, CE), jnp.float32),
            pltpu.VMEM_SHARED((2, CE), jnp.float32),
            pltpu.SemaphoreType.DMA,
            pltpu.SemaphoreType.DMA,
        ],
    )
    def k(qi_h, ui_h, vi_h, ci_h, Qt, Ut, Ct, Vt, Pt,
          oq, ou, oc, ov, opos, idxbuf, r32, r16, posb, pv, c_sh, sem_g, sem_s):
        wid = lax.axis_index("s") * NC + lax.axis_index("c")

        with jax.named_scope("q_gather"):
            _do_table(wid, qi_h, Qt, oq, idxbuf, r32, sem_g, sem_s,
                      rows_w, L, per_w)
        with jax.named_scope("u_gather"):
            _do_table(wid, ui_h, Ut, ou, idxbuf, r32, sem_g, sem_s,
                      rows_w, L, per_w)
        with jax.named_scope("v_gather"):
            _do_table(wid, vi_h, Vt, ov, idxbuf, r16, sem_g, sem_s,
                      rows_w, L, per_w)

        with jax.named_scope("c_gather"):
            sid = lax.axis_index("s")

            @pl.when(sid == 0)
            def _():
                pltpu.sync_copy(Ct, c_sh)

            plsc.subcore_barrier()
            _do_table(wid, ci_h, c_sh, oc, idxbuf, r16, sem_g, sem_s,
                      rows_w, L, per_w)

        with jax.named_scope("pos_store"):
            pltpu.sync_copy(Pt, pv)
            prow = [pv[i, :] for i in range(10)]
            for b in range(POS_ROWS // 10):
                for r in range(10):
                    posb[b * 10 + r, :] = prow[r]

            nstores = per_w // POS_ROWS

            def pos_body(t, carry):
                cps = []
                for u in range(POS_Q):
                    base = wid * per_w + (t * POS_Q + u) * POS_ROWS
                    cps.append(
                        pltpu.async_copy(
                            posb, opos.at[pl.ds(base, POS_ROWS)], sem_s
                        )
                    )
                for cp in cps:
                    cp.wait()
                return carry

            lax.fori_loop(0, nstores // POS_Q, pos_body, 0)

    oq, ou, oc, ov, opos = k(qids, uids, vids, clicks,
                             Q_table, U_table, C_table, V_table, P_table)
    return (
        oq.reshape(B, L, E),
        ou.reshape(B, L, E),
        oc.reshape(B, L, CE),
        ov.reshape(B, L, CE),
        opos.reshape(B, L, CE),
    )
